# baseline (speedup 1.0000x reference)
import functools

import jax
import jax.numpy as jnp
from jax import lax
from jax.experimental import pallas as pl
from jax.experimental.pallas import tpu as pltpu
from jax.experimental.pallas import tpu_sc as plsc

_NC = 2
_NS = 16
_NW = _NC * _NS


def _widen_rows_tc(table):
  v, d = table.shape
  blk = 1000
  assert v % blk == 0

  def body(in_ref, out_ref):
    out_ref[:, :d] = in_ref[...]
    out_ref[:, d:] = jnp.zeros((blk, d), jnp.float32)

  return pl.pallas_call(
      body,
      grid=(v // blk,),
      in_specs=[pl.BlockSpec((blk, d), lambda i: (i, 0))],
      out_specs=pl.BlockSpec((blk, 2 * d), lambda i: (i, 0)),
      out_shape=jax.ShapeDtypeStruct((v, 2 * d), jnp.float32),
  )(table)


def _make_gather(b: int, l: int, d: int):
  grp = l // 2
  assert grp <= 128
  b_per_w = b // _NW
  g_per_w = 2 * b_per_w
  mesh = plsc.VectorSubcoreMesh(core_axis_name="c", subcore_axis_name="s")

  @functools.partial(
      pl.kernel,
      out_type=jax.ShapeDtypeStruct((b, l, d), jnp.float32),
      mesh=mesh,
      scratch_types=[
          pltpu.VMEM((g_per_w, grp), jnp.int32),
          pltpu.VMEM((2, grp, 2 * d), jnp.float32),
          pltpu.VMEM((2, l, d), jnp.float32),
          pltpu.SemaphoreType.DMA((2,)),
          pltpu.SemaphoreType.DMA((2,)),
      ],
      compiler_params=pltpu.CompilerParams(use_tc_tiling_on_sc=True),
  )
  def gather_kernel(t2_hbm, idx_hbm, out_hbm, idx_v, gbuf, pbuf, in_sem,
                    out_sem):
    wid = lax.axis_index("s") * _NC + lax.axis_index("c")
    gbase = wid * g_per_w
    bbase = wid * b_per_w
    pltpu.sync_copy(idx_hbm.at[pl.ds(gbase, g_per_w)], idx_v)

    def bridge(gb, pb, h):
      @pl.loop(0, grp, unroll=4)
      def _row(i):
        for c in range(d // 16):
          pbuf[pb, h * grp + i, pl.ds(c * 16, 16)] = (
              gbuf[gb, i, pl.ds(c * 16, 16)])

    for j in range(2):
      pltpu.async_copy(t2_hbm.at[idx_v.at[j]], gbuf.at[j], in_sem.at[j])

    @pl.loop(0, g_per_w, step=4)
    def _outer(t0):
      for j in range(4):
        t = t0 + j
        gb = j % 2
        pb = (j // 2) % 2
        h = j % 2
        bidx = bbase + (t0 + j) // 2
        pltpu.make_async_copy(t2_hbm.at[idx_v.at[t]], gbuf.at[gb],
                              in_sem.at[gb]).wait()
        if h == 0:
          @pl.when(t >= 4)
          def _():
            pltpu.make_async_copy(pbuf.at[pb], out_hbm.at[bidx],
                                  out_sem.at[pb]).wait()

        bridge(gb, pb, h)

        @pl.when(t + 2 < g_per_w)
        def _():
          pltpu.async_copy(t2_hbm.at[idx_v.at[t + 2]], gbuf.at[gb],
                           in_sem.at[gb])

        if h == 1:
          pltpu.async_copy(pbuf.at[pb], out_hbm.at[bidx], out_sem.at[pb])

    for pb in range(2):
      bidx = bbase + b_per_w - 2 + pb
      pltpu.make_async_copy(pbuf.at[pb], out_hbm.at[bidx],
                            out_sem.at[pb]).wait()

  return gather_kernel


def kernel(x, mask_ids, table):
  del mask_ids
  b, l = x.shape
  _, d = table.shape
  t2 = _widen_rows_tc(table)
  idx = x.reshape(b * l // (l // 2), l // 2).astype(jnp.int32)
  out = _make_gather(b, l, d)(t2, idx)
  return out, jnp.asarray(0.0, dtype=jnp.float32)


# speedup vs baseline: 1.4600x; 1.4600x over previous
import functools

import jax
import jax.numpy as jnp
from jax import lax
from jax.experimental import pallas as pl
from jax.experimental.pallas import tpu as pltpu
from jax.experimental.pallas import tpu_sc as plsc

_NC = 2
_NS = 16
_NW = _NC * _NS


def _widen_rows_tc(table):
  v, d = table.shape
  blk = 1000
  assert v % blk == 0

  def body(in_ref, out_ref):
    out_ref[:, :d] = in_ref[...]
    out_ref[:, d:] = jnp.zeros((blk, d), jnp.float32)

  return pl.pallas_call(
      body,
      grid=(v // blk,),
      in_specs=[pl.BlockSpec((blk, d), lambda i: (i, 0))],
      out_specs=pl.BlockSpec((blk, 2 * d), lambda i: (i, 0)),
      out_shape=jax.ShapeDtypeStruct((v, 2 * d), jnp.float32),
  )(table)


def _make_gather(b: int, l: int, d: int):
  grp = l // 2
  assert grp <= 128
  b_per_w = b // _NW
  g_per_w = 2 * b_per_w
  mesh = plsc.VectorSubcoreMesh(core_axis_name="c", subcore_axis_name="s")

  @functools.partial(
      pl.kernel,
      out_type=jax.ShapeDtypeStruct((b, l, d), jnp.float32),
      mesh=mesh,
      scratch_types=[
          pltpu.VMEM((g_per_w, grp), jnp.int32),
          pltpu.VMEM((2, grp, 2 * d), jnp.float32),
          pltpu.VMEM((2, l, d), jnp.float32),
          pltpu.SemaphoreType.DMA((2,)),
          pltpu.SemaphoreType.DMA((2,)),
      ],
      compiler_params=pltpu.CompilerParams(use_tc_tiling_on_sc=True),
  )
  def gather_kernel(t2_hbm, idx_hbm, out_hbm, idx_v, gbuf, pbuf, in_sem,
                    out_sem):
    wid = lax.axis_index("s") * _NC + lax.axis_index("c")
    gbase = wid * g_per_w
    bbase = wid * b_per_w
    pltpu.sync_copy(idx_hbm.at[pl.ds(gbase, g_per_w)], idx_v)

    def bridge(gb, pb, h):
      @pl.loop(0, grp, unroll=4)
      def _row(i):
        for c in range(d // 16):
          pbuf[pb, h * grp + i, pl.ds(c * 16, 16)] = (
              gbuf[gb, i, pl.ds(c * 16, 16)])

    for j in range(2):
      pltpu.async_copy(t2_hbm.at[idx_v.at[j]], gbuf.at[j], in_sem.at[j])

    @pl.loop(0, g_per_w, step=4)
    def _outer(t0):
      for j in range(4):
        t = t0 + j
        gb = j % 2
        pb = (j // 2) % 2
        h = j % 2
        bidx = bbase + (t0 + j) // 2
        pltpu.make_async_copy(t2_hbm.at[idx_v.at[t]], gbuf.at[gb],
                              in_sem.at[gb]).wait()
        if h == 0:
          @pl.when(t >= 4)
          def _():
            pltpu.make_async_copy(pbuf.at[pb], out_hbm.at[bidx],
                                  out_sem.at[pb]).wait()

        bridge(gb, pb, h)

        @pl.when(t + 2 < g_per_w)
        def _():
          pltpu.async_copy(t2_hbm.at[idx_v.at[t + 2]], gbuf.at[gb],
                           in_sem.at[gb])

        if h == 1:
          pltpu.async_copy(pbuf.at[pb], out_hbm.at[bidx], out_sem.at[pb])

    for pb in range(2):
      bidx = bbase + b_per_w - 2 + pb
      pltpu.make_async_copy(pbuf.at[pb], out_hbm.at[bidx],
                            out_sem.at[pb]).wait()

  return gather_kernel


def kernel(x, mask_ids, table):
  del mask_ids
  b, l = x.shape
  _, d = table.shape
  t2 = jnp.pad(table, ((0, 0), (0, d)))
  idx = x.reshape(b * l // (l // 2), l // 2).astype(jnp.int32)
  out = _make_gather(b, l, d)(t2, idx)
  return out, jnp.asarray(0.0, dtype=jnp.float32)


# speedup vs baseline: 1.9423x; 1.3303x over previous
import functools

import jax
import jax.numpy as jnp
from jax import lax
from jax.experimental import pallas as pl
from jax.experimental.pallas import tpu as pltpu
from jax.experimental.pallas import tpu_sc as plsc

_NC = 2
_NS = 16
_NW = _NC * _NS


def _widen_rows_tc(table):
  v, d = table.shape
  blk = 1000
  assert v % blk == 0

  def body(in_ref, out_ref):
    out_ref[:, :d] = in_ref[...]
    out_ref[:, d:] = jnp.zeros((blk, d), jnp.float32)

  return pl.pallas_call(
      body,
      grid=(v // blk,),
      in_specs=[pl.BlockSpec((blk, d), lambda i: (i, 0))],
      out_specs=pl.BlockSpec((blk, 2 * d), lambda i: (i, 0)),
      out_shape=jax.ShapeDtypeStruct((v, 2 * d), jnp.float32),
  )(table)


def _make_gather(b: int, l: int, d: int):
  grp = l // 2
  assert grp <= 128
  b_per_w = b // _NW
  g_per_w = 2 * b_per_w
  mesh = plsc.VectorSubcoreMesh(core_axis_name="c", subcore_axis_name="s")

  @functools.partial(
      pl.kernel,
      out_type=jax.ShapeDtypeStruct((b, l, d), jnp.float32),
      mesh=mesh,
      scratch_types=[
          pltpu.VMEM((g_per_w, grp), jnp.int32),
          pltpu.VMEM((2, grp, 2 * d), jnp.float32),
          pltpu.VMEM((2, l, d), jnp.float32),
          pltpu.SemaphoreType.DMA((2,)),
          pltpu.SemaphoreType.DMA((2,)),
      ],
      compiler_params=pltpu.CompilerParams(use_tc_tiling_on_sc=True),
  )
  def gather_kernel(t2_hbm, idx_hbm, out_hbm, idx_v, gbuf, pbuf, in_sem,
                    out_sem):
    wid = lax.axis_index("s") * _NC + lax.axis_index("c")
    gbase = wid * g_per_w
    bbase = wid * b_per_w
    pltpu.sync_copy(idx_hbm.at[pl.ds(gbase, g_per_w)], idx_v)

    def bridge(gb, pb, h):
      @pl.loop(0, grp, unroll=4)
      def _row(i):
        for c in range(d // 16):
          pbuf[pb, h * grp + i, pl.ds(c * 16, 16)] = (
              gbuf[gb, i, pl.ds(c * 16, 16)])

    for j in range(2):
      pltpu.async_copy(t2_hbm.at[idx_v.at[j]], gbuf.at[j], in_sem.at[j])

    @pl.loop(0, g_per_w, step=4)
    def _outer(t0):
      for j in range(4):
        t = t0 + j
        gb = j % 2
        pb = (j // 2) % 2
        h = j % 2
        bidx = bbase + (t0 + j) // 2
        pltpu.make_async_copy(t2_hbm.at[idx_v.at[t]], gbuf.at[gb],
                              in_sem.at[gb]).wait()
        if h == 0:
          @pl.when(t >= 4)
          def _():
            pltpu.make_async_copy(pbuf.at[pb], out_hbm.at[bidx],
                                  out_sem.at[pb]).wait()

        bridge(gb, pb, h)

        @pl.when(t + 2 < g_per_w)
        def _():
          pltpu.async_copy(t2_hbm.at[idx_v.at[t + 2]], gbuf.at[gb],
                           in_sem.at[gb])

        if h == 1:
          pltpu.async_copy(pbuf.at[pb], out_hbm.at[bidx], out_sem.at[pb])

    for pb in range(2):
      bidx = bbase + b_per_w - 2 + pb
      pltpu.make_async_copy(pbuf.at[pb], out_hbm.at[bidx],
                            out_sem.at[pb]).wait()

  return gather_kernel


def kernel(x, mask_ids, table):
  del mask_ids
  b, l = x.shape
  _, d = table.shape
  eye_wide = jnp.eye(d, 2 * d, dtype=jnp.float32)
  t2 = table @ eye_wide
  idx = x.reshape(b * l // (l // 2), l // 2).astype(jnp.int32)
  out = _make_gather(b, l, d)(t2, idx)
  return out, jnp.asarray(0.0, dtype=jnp.float32)
